# baseline (speedup 1.0000x reference)
import functools

import jax
import jax.numpy as jnp
from jax import lax
from jax.experimental import pallas as pl
from jax.experimental.pallas import tpu as pltpu
from jax.experimental.pallas import tpu_sc as plsc

_N = 10000
_NPAD = 10240
_E = 320000
_D = 128
_CHUNK = 128
_NCHUNKS = _E // _CHUNK
_NW = 32
_RPT = _NPAD // 16

_NB = 2
_CPT = 80
_NCHUNKS_P = _CPT * _NW
_EPAD = _NCHUNKS_P * _CHUNK - _E

_mesh = plsc.VectorSubcoreMesh(core_axis_name="c", subcore_axis_name="s")


@functools.partial(
    pl.kernel,
    out_type=jax.ShapeDtypeStruct((2, _NPAD, _D), jnp.float32),
    mesh=_mesh,
    scratch_types=[
        pltpu.VMEM_SHARED((_NPAD, _D), jnp.float32),
        pltpu.VMEM((_NB, _CHUNK, _D), jnp.float32),
        pltpu.VMEM((_CPT, _CHUNK), jnp.int32),
        pltpu.VMEM((_NB, _CHUNK), jnp.int32),
        pltpu.VMEM((1, _CHUNK), jnp.int32),
        pltpu.SemaphoreType.DMA((_NB,)),
        pltpu.SemaphoreType.DMA,
    ],
)
def _agg_kernel(hs_hbm, combo_hbm, zrows_hbm, out_hbm,
                acc, rows, combo, sidx, didx, gsem, ssem):
    cid = lax.axis_index("c")
    sid = lax.axis_index("s")
    base = sid * _RPT
    pltpu.sync_copy(zrows_hbm, acc.at[pl.ds(base, _RPT)])
    wid = sid * 2 + cid
    pltpu.sync_copy(combo_hbm.at[pl.ds(wid * _CPT, _CPT)], combo)
    plsc.subcore_barrier()

    def fire_gather(j):
        slot = lax.rem(j, _NB)
        for k in range(_CHUNK // 16):
            v = combo[j, pl.ds(k * 16, 16)]
            sidx[slot, pl.ds(k * 16, 16)] = lax.bitwise_and(v, 0xFFFF)
        pltpu.async_copy(hs_hbm.at[sidx.at[slot]], rows.at[slot], gsem.at[slot])

    def prologue(j, carry):
        fire_gather(j)
        return carry

    lax.fori_loop(0, _NB, prologue, 0)

    def body(j, carry):
        slot = lax.rem(j, _NB)
        pltpu.make_async_copy(hs_hbm.at[sidx.at[slot]], rows.at[slot],
                              gsem.at[slot]).wait()
        for k in range(_CHUNK // 16):
            didx[0, pl.ds(k * 16, 16)] = lax.shift_right_logical(
                combo[j, pl.ds(k * 16, 16)], 16)
        pltpu.async_copy(rows.at[slot], acc.at[didx.at[0]], ssem,
                         add=True).wait()

        @pl.when(j + _NB < _CPT)
        def _():
            fire_gather(j + _NB)

        return carry

    lax.fori_loop(0, _CPT, body, 0)
    plsc.subcore_barrier()
    pltpu.sync_copy(acc.at[pl.ds(base, _RPT)], out_hbm.at[cid, pl.ds(base, _RPT)])


_R = 1000


def _tc_pre_body(deg_ref, x_ref, w_ref, hs_ref, dis_ref):
    deg = deg_ref[0, :, :1] + deg_ref[1, :, :1] + 1.0
    dis = lax.rsqrt(deg)
    dis_ref[...] = jnp.broadcast_to(dis, (_R, 16))
    xw = jnp.dot(x_ref[...], w_ref[...], preferred_element_type=jnp.float32)
    hs_ref[...] = xw * dis


def _tc_pre(degpair, x, W0):
    return pl.pallas_call(
        _tc_pre_body,
        grid=(_N // _R,),
        in_specs=[
            pl.BlockSpec((2, _R, _D), lambda i: (0, i, 0)),
            pl.BlockSpec((_R, _D), lambda i: (i, 0)),
            pl.BlockSpec((_D, _D), lambda i: (0, 0)),
        ],
        out_specs=[
            pl.BlockSpec((_R, _D), lambda i: (i, 0)),
            pl.BlockSpec((_R, 16), lambda i: (i, 0)),
        ],
        out_shape=[
            jax.ShapeDtypeStruct((_N, _D), jnp.float32),
            jax.ShapeDtypeStruct((_N, 16), jnp.float32),
        ],
    )(degpair, x, W0)


def _tc_mid_body(p_ref, hs_ref, dis_ref, b_ref, w_ref, o_ref):
    d = dis_ref[:, :1]
    agg = (p_ref[0] + p_ref[1] + hs_ref[...]) * d
    h = jnp.maximum(agg + b_ref[...], 0.0)
    o_ref[...] = jnp.dot(h, w_ref[...], preferred_element_type=jnp.float32) * d


def _tc_mid(p, hs, dis, b, W):
    return pl.pallas_call(
        _tc_mid_body,
        grid=(_N // _R,),
        in_specs=[
            pl.BlockSpec((2, _R, _D), lambda i: (0, i, 0)),
            pl.BlockSpec((_R, _D), lambda i: (i, 0)),
            pl.BlockSpec((_R, 16), lambda i: (i, 0)),
            pl.BlockSpec((1, _D), lambda i: (0, 0)),
            pl.BlockSpec((_D, _D), lambda i: (0, 0)),
        ],
        out_specs=pl.BlockSpec((_R, _D), lambda i: (i, 0)),
        out_shape=jax.ShapeDtypeStruct((_N, _D), jnp.float32),
    )(p, hs, dis, b.reshape(1, _D), W)


def _tc_fin_body(p_ref, hs_ref, dis_ref, b_ref, o_ref):
    d = dis_ref[:, :1]
    z = (p_ref[0] + p_ref[1] + hs_ref[...]) * d + b_ref[...]
    m = jnp.max(z, axis=1, keepdims=True)
    zs = z - m
    o_ref[...] = zs - jnp.log(jnp.sum(jnp.exp(zs), axis=1, keepdims=True))


def _tc_fin(p, hs, dis, b):
    return pl.pallas_call(
        _tc_fin_body,
        grid=(_N // _R,),
        in_specs=[
            pl.BlockSpec((2, _R, _D), lambda i: (0, i, 0)),
            pl.BlockSpec((_R, _D), lambda i: (i, 0)),
            pl.BlockSpec((_R, 16), lambda i: (i, 0)),
            pl.BlockSpec((1, _D), lambda i: (0, 0)),
        ],
        out_specs=pl.BlockSpec((_R, _D), lambda i: (i, 0)),
        out_shape=jax.ShapeDtypeStruct((_N, _D), jnp.float32),
    )(p, hs, dis, b.reshape(1, _D))


def kernel(x, edge_index, W0, b0, W1, b1, W2, b2):
    src = jnp.concatenate([edge_index[0], jnp.zeros((_EPAD,), jnp.int32)])
    dst = jnp.concatenate([edge_index[1], jnp.full((_EPAD,), _N, jnp.int32)])
    combo = (jnp.left_shift(dst, 16) | src).reshape(_NCHUNKS_P, _CHUNK)
    zrows = jnp.zeros((_RPT, _D), jnp.float32)
    ones_tbl = jnp.ones((_N, _D), jnp.float32)

    degpair = _agg_kernel(ones_tbl, combo, zrows)
    hs0, dis = _tc_pre(degpair, x, W0)
    p0 = _agg_kernel(hs0, combo, zrows)
    hs1 = _tc_mid(p0, hs0, dis, b0, W1)
    p1 = _agg_kernel(hs1, combo, zrows)
    hs2 = _tc_mid(p1, hs1, dis, b1, W2)
    p2 = _agg_kernel(hs2, combo, zrows)
    return _tc_fin(p2, hs2, dis, b2)


# speedup vs baseline: 1.1160x; 1.1160x over previous
import functools

import jax
import jax.numpy as jnp
from jax import lax
from jax.experimental import pallas as pl
from jax.experimental.pallas import tpu as pltpu
from jax.experimental.pallas import tpu_sc as plsc

_N = 10000
_NPAD = 10240
_E = 320000
_D = 128
_CHUNK = 128
_NCHUNKS = _E // _CHUNK
_NW = 32
_RPT = _NPAD // 16

_NB = 2
_CPT = 80
_NCHUNKS_P = _CPT * _NW
_EPAD = _NCHUNKS_P * _CHUNK - _E

_mesh = plsc.VectorSubcoreMesh(core_axis_name="c", subcore_axis_name="s")


@functools.partial(
    pl.kernel,
    out_type=jax.ShapeDtypeStruct((2, _NPAD, _D), jnp.float32),
    mesh=_mesh,
    scratch_types=[
        pltpu.VMEM_SHARED((_NPAD, _D), jnp.float32),
        pltpu.VMEM((_NB, _CHUNK, _D), jnp.float32),
        pltpu.VMEM((2 * _NB, _CHUNK), jnp.int32),
        pltpu.VMEM((2 * _NB, _CHUNK), jnp.int32),
        pltpu.SemaphoreType.DMA((_NB,)),
        pltpu.SemaphoreType.DMA((2 * _NB,)),
        pltpu.SemaphoreType.DMA((2 * _NB,)),
        pltpu.SemaphoreType.DMA,
    ],
)
def _agg_kernel(hs_hbm, src2d_hbm, dst2d_hbm, zrows_hbm, out_hbm,
                acc, rows, sidx, didx, gsem, isem, dsem, ssem):
    cid = lax.axis_index("c")
    sid = lax.axis_index("s")
    base = sid * _RPT
    pltpu.sync_copy(zrows_hbm, acc.at[pl.ds(base, _RPT)])
    wid = sid * 2 + cid
    c0 = wid * _CPT
    plsc.subcore_barrier()

    def fire_idx(j, si):
        pltpu.async_copy(src2d_hbm.at[c0 + j], sidx.at[si], isem.at[si])
        pltpu.async_copy(dst2d_hbm.at[c0 + j], didx.at[si], dsem.at[si])

    def wait_idx(j, si):
        pltpu.make_async_copy(src2d_hbm.at[c0 + j], sidx.at[si],
                              isem.at[si]).wait()
        pltpu.make_async_copy(dst2d_hbm.at[c0 + j], didx.at[si],
                              dsem.at[si]).wait()

    def fire_gather(si, rs):
        pltpu.async_copy(hs_hbm.at[sidx.at[si]], rows.at[rs], gsem.at[rs])

    def wait_gather(si, rs):
        pltpu.make_async_copy(hs_hbm.at[sidx.at[si]], rows.at[rs],
                              gsem.at[rs]).wait()

    def scatter(si, rs):
        pltpu.async_copy(rows.at[rs], acc.at[didx.at[si]], ssem,
                         add=True).wait()

    for t in range(2 * _NB):
        fire_idx(t, t)
    for t in range(_NB):
        wait_idx(t, t)
        fire_gather(t, t)

    def body(j2, carry):
        for t in range(2 * _NB):
            j = j2 * 2 * _NB + t
            rs = t % _NB
            wait_gather(t, rs)
            scatter(t, rs)

            @pl.when(j + 2 * _NB < _CPT)
            def _():
                fire_idx(j + 2 * _NB, t)

            @pl.when(j + _NB < _CPT)
            def _():
                wait_idx(j + _NB, (t + _NB) % (2 * _NB))
                fire_gather((t + _NB) % (2 * _NB), rs)

        return carry

    lax.fori_loop(0, _CPT // (2 * _NB), body, 0)
    plsc.subcore_barrier()
    pltpu.sync_copy(acc.at[pl.ds(base, _RPT)], out_hbm.at[cid, pl.ds(base, _RPT)])


_R = 1000


def _tc_pre_body(deg_ref, x_ref, w_ref, hs_ref, dis_ref):
    deg = deg_ref[0, :, :1] + deg_ref[1, :, :1] + 1.0
    dis = lax.rsqrt(deg)
    dis_ref[...] = jnp.broadcast_to(dis, (_R, 16))
    xw = jnp.dot(x_ref[...], w_ref[...], preferred_element_type=jnp.float32)
    hs_ref[...] = xw * dis


def _tc_pre(degpair, x, W0):
    return pl.pallas_call(
        _tc_pre_body,
        grid=(_N // _R,),
        in_specs=[
            pl.BlockSpec((2, _R, _D), lambda i: (0, i, 0)),
            pl.BlockSpec((_R, _D), lambda i: (i, 0)),
            pl.BlockSpec((_D, _D), lambda i: (0, 0)),
        ],
        out_specs=[
            pl.BlockSpec((_R, _D), lambda i: (i, 0)),
            pl.BlockSpec((_R, 16), lambda i: (i, 0)),
        ],
        out_shape=[
            jax.ShapeDtypeStruct((_N, _D), jnp.float32),
            jax.ShapeDtypeStruct((_N, 16), jnp.float32),
        ],
    )(degpair, x, W0)


def _tc_mid_body(p_ref, hs_ref, dis_ref, b_ref, w_ref, o_ref):
    d = dis_ref[:, :1]
    agg = (p_ref[0] + p_ref[1] + hs_ref[...]) * d
    h = jnp.maximum(agg + b_ref[...], 0.0)
    o_ref[...] = jnp.dot(h, w_ref[...], preferred_element_type=jnp.float32) * d


def _tc_mid(p, hs, dis, b, W):
    return pl.pallas_call(
        _tc_mid_body,
        grid=(_N // _R,),
        in_specs=[
            pl.BlockSpec((2, _R, _D), lambda i: (0, i, 0)),
            pl.BlockSpec((_R, _D), lambda i: (i, 0)),
            pl.BlockSpec((_R, 16), lambda i: (i, 0)),
            pl.BlockSpec((1, _D), lambda i: (0, 0)),
            pl.BlockSpec((_D, _D), lambda i: (0, 0)),
        ],
        out_specs=pl.BlockSpec((_R, _D), lambda i: (i, 0)),
        out_shape=jax.ShapeDtypeStruct((_N, _D), jnp.float32),
    )(p, hs, dis, b.reshape(1, _D), W)


def _tc_fin_body(p_ref, hs_ref, dis_ref, b_ref, o_ref):
    d = dis_ref[:, :1]
    z = (p_ref[0] + p_ref[1] + hs_ref[...]) * d + b_ref[...]
    m = jnp.max(z, axis=1, keepdims=True)
    zs = z - m
    o_ref[...] = zs - jnp.log(jnp.sum(jnp.exp(zs), axis=1, keepdims=True))


def _tc_fin(p, hs, dis, b):
    return pl.pallas_call(
        _tc_fin_body,
        grid=(_N // _R,),
        in_specs=[
            pl.BlockSpec((2, _R, _D), lambda i: (0, i, 0)),
            pl.BlockSpec((_R, _D), lambda i: (i, 0)),
            pl.BlockSpec((_R, 16), lambda i: (i, 0)),
            pl.BlockSpec((1, _D), lambda i: (0, 0)),
        ],
        out_specs=pl.BlockSpec((_R, _D), lambda i: (i, 0)),
        out_shape=jax.ShapeDtypeStruct((_N, _D), jnp.float32),
    )(p, hs, dis, b.reshape(1, _D))


def kernel(x, edge_index, W0, b0, W1, b1, W2, b2):
    src = jnp.concatenate(
        [edge_index[0], jnp.zeros((_EPAD,), jnp.int32)]
    ).reshape(_NCHUNKS_P, _CHUNK)
    dst = jnp.concatenate(
        [edge_index[1], jnp.full((_EPAD,), _N, jnp.int32)]
    ).reshape(_NCHUNKS_P, _CHUNK)
    zrows = jnp.zeros((_RPT, _D), jnp.float32)
    ones_tbl = jnp.ones((_N, _D), jnp.float32)

    degpair = _agg_kernel(ones_tbl, src, dst, zrows)
    hs0, dis = _tc_pre(degpair, x, W0)
    p0 = _agg_kernel(hs0, src, dst, zrows)
    hs1 = _tc_mid(p0, hs0, dis, b0, W1)
    p1 = _agg_kernel(hs1, src, dst, zrows)
    hs2 = _tc_mid(p1, hs1, dis, b1, W2)
    p2 = _agg_kernel(hs2, src, dst, zrows)
    return _tc_fin(p2, hs2, dis, b2)
